# baseline (speedup 1.0000x reference)
import jax
import jax.numpy as jnp
from jax import lax
from jax.experimental import pallas as pl
from jax.experimental.pallas import tpu as pltpu
from jax.experimental.pallas import tpu_sc as plsc

N_EDGE = 160000
N_NODE = 10000
D = 256
DW = D // 2
NC = 2
NS = 16
NW = NC * NS
PER_W = N_EDGE // NW
C = 64
N_FULL = PER_W // C
TAIL_OFF = PER_W - C
NCH = N_FULL + 2
LANES = 16


def _sc_body(zw_hbm, src_hbm, dst_hbm, out_hbm,
             srcv, dstv, out_full, zshared,
             rows_s0, rows_d0, rows_s1, rows_d1,
             sem_s0, sem_d0, sem_s1, sem_d1):
    sid = lax.axis_index("s")
    wid = sid * NC + lax.axis_index("c")
    wbase = wid * PER_W
    rows_s = (rows_s0, rows_s1)
    rows_d = (rows_d0, rows_d1)
    sem_s = (sem_s0, sem_s1)
    sem_d = (sem_d0, sem_d1)

    pltpu.sync_copy(zw_hbm.at[pl.ds(sid * (N_NODE // NS), N_NODE // NS)],
                    zshared.at[pl.ds(sid * (N_NODE // NS), N_NODE // NS)])
    pltpu.sync_copy(src_hbm.at[pl.ds(wbase, PER_W)], srcv)
    pltpu.sync_copy(dst_hbm.at[pl.ds(wbase, PER_W)], dstv)
    plsc.subcore_barrier()

    def chunk_off(c):
        return jnp.where(c < N_FULL, c * C, TAIL_OFF)

    def issue(c, b):
        off = chunk_off(c)
        pltpu.async_copy(zshared.at[srcv.at[pl.ds(off, C)]], rows_s[b],
                         sem_s[b])
        pltpu.async_copy(zshared.at[dstv.at[pl.ds(off, C)]], rows_d[b],
                         sem_d[b])

    def drain(c, b):
        off = chunk_off(c)
        pltpu.make_async_copy(zshared.at[srcv.at[pl.ds(off, C)]], rows_s[b],
                              sem_s[b]).wait()
        pltpu.make_async_copy(zshared.at[dstv.at[pl.ds(off, C)]], rows_d[b],
                              sem_d[b]).wait()

    for b in range(2):
        issue(b, b)

    def pair_body(i, carry):
        for b in range(2):
            c = i * 2 + b
            drain(c, b)
            rs, rd = rows_s[b], rows_d[b]
            off = chunk_off(c)

            def group_body(g, carry2, _rs=rs, _rd=rd, _off=off):
                lanes = g * LANES + lax.iota(jnp.int32, LANES)

                def word_body(kb, carry3):
                    kv, f0, f1, f2, f3 = carry3
                    pz = jnp.zeros((2 * LANES,), jnp.bfloat16)
                    pacc = [pz, pz]
                    for j in range(16):
                        ws = plsc.load_gather(_rs, [lanes, kv])
                        wd = plsc.load_gather(_rd, [lanes, kv])
                        p = (plsc.bitcast(ws, jnp.bfloat16) *
                             plsc.bitcast(wd, jnp.bfloat16))
                        pacc[j % 2] = pacc[j % 2] + p
                        kv = (kv + 1) & (DW - 1)
                    a0, b0 = plsc.unpack(pacc[0],
                                         format=plsc.PackFormat.INTERLEAVED)
                    a1, b1 = plsc.unpack(pacc[1],
                                         format=plsc.PackFormat.INTERLEAVED)
                    return (kv, f0 + a0, f1 + b0, f2 + a1, f3 + b1)

                zero = jnp.zeros((LANES,), jnp.float32)
                kv0 = lax.iota(jnp.int32, LANES) & (DW - 1)
                carry3 = lax.fori_loop(
                    0, DW // 16, word_body, (kv0, zero, zero, zero, zero))
                _, f0, f1, f2, f3 = carry3
                out_full[pl.ds(_off + g * LANES, LANES)] = (f0 + f1) + (f2 + f3)
                return carry2

            lax.fori_loop(0, C // LANES, group_body, 0)

            @pl.when(c + 2 < NCH)
            def _():
                issue(c + 2, b)

        return carry

    lax.fori_loop(0, (NCH + 1) // 2, pair_body, 0)
    pltpu.sync_copy(out_full, out_hbm.at[pl.ds(wbase, PER_W)])


@jax.jit
def _link_predict(zw, src, dst):
    mesh = plsc.VectorSubcoreMesh(core_axis_name="c", subcore_axis_name="s")
    run = pl.kernel(
        _sc_body,
        out_type=jax.ShapeDtypeStruct((N_EDGE,), jnp.float32),
        mesh=mesh,
        scratch_types=[
            pltpu.VMEM((PER_W,), jnp.int32),
            pltpu.VMEM((PER_W,), jnp.int32),
            pltpu.VMEM((PER_W,), jnp.float32),
            pltpu.VMEM_SHARED((N_NODE, DW), jnp.int32),
            pltpu.VMEM((C, DW), jnp.int32),
            pltpu.VMEM((C, DW), jnp.int32),
            pltpu.VMEM((C, DW), jnp.int32),
            pltpu.VMEM((C, DW), jnp.int32),
            pltpu.SemaphoreType.DMA,
            pltpu.SemaphoreType.DMA,
            pltpu.SemaphoreType.DMA,
            pltpu.SemaphoreType.DMA,
        ],
        compiler_params=pltpu.CompilerParams(
            use_tc_tiling_on_sc=False, needs_layout_passes=False),
    )
    return run(zw, src, dst)


def kernel(z, edge):
    src = edge[:, 0].astype(jnp.int32)
    dst = edge[:, 1].astype(jnp.int32)
    zw = jax.lax.bitcast_convert_type(
        z.astype(jnp.bfloat16).reshape(N_NODE, DW, 2), jnp.int32)
    return _link_predict(zw, src, dst)


# speedup vs baseline: 1.0287x; 1.0287x over previous
import jax
import jax.numpy as jnp
from jax import lax
from jax.experimental import pallas as pl
from jax.experimental.pallas import tpu as pltpu
from jax.experimental.pallas import tpu_sc as plsc

N_EDGE = 160000
N_NODE = 10000
D = 256
DW = D // 2
NC = 2
NS = 16
NW = NC * NS
PER_W = N_EDGE // NW
C = 64
N_FULL = PER_W // C
TAIL_OFF = PER_W - C
NCH = N_FULL + 2
LANES = 16


def _sc_body(zw_hbm, src_hbm, dst_hbm, out_hbm,
             srcv, dstv, out_full, zshared,
             rows_s0, rows_d0, rows_s1, rows_d1,
             sem_s0, sem_d0, sem_s1, sem_d1):
    sid = lax.axis_index("s")
    wid = sid * NC + lax.axis_index("c")
    wbase = wid * PER_W
    rows_s = (rows_s0, rows_s1)
    rows_d = (rows_d0, rows_d1)
    sem_s = (sem_s0, sem_s1)
    sem_d = (sem_d0, sem_d1)

    pltpu.sync_copy(zw_hbm.at[pl.ds(sid * (N_NODE // NS), N_NODE // NS)],
                    zshared.at[pl.ds(sid * (N_NODE // NS), N_NODE // NS)])
    pltpu.sync_copy(src_hbm.at[pl.ds(wbase, PER_W)], srcv)
    pltpu.sync_copy(dst_hbm.at[pl.ds(wbase, PER_W)], dstv)
    plsc.subcore_barrier()

    def chunk_off(c):
        return jnp.where(c < N_FULL, c * C, TAIL_OFF)

    def issue(c, b):
        off = chunk_off(c)
        pltpu.async_copy(zw_hbm.at[srcv.at[pl.ds(off, C)]], rows_s[b],
                         sem_s[b])
        pltpu.async_copy(zshared.at[dstv.at[pl.ds(off, C)]], rows_d[b],
                         sem_d[b])

    def drain(c, b):
        off = chunk_off(c)
        pltpu.make_async_copy(zw_hbm.at[srcv.at[pl.ds(off, C)]], rows_s[b],
                              sem_s[b]).wait()
        pltpu.make_async_copy(zshared.at[dstv.at[pl.ds(off, C)]], rows_d[b],
                              sem_d[b]).wait()

    for b in range(2):
        issue(b, b)

    def pair_body(i, carry):
        for b in range(2):
            c = i * 2 + b
            drain(c, b)
            rs, rd = rows_s[b], rows_d[b]
            off = chunk_off(c)

            def group_body(g, carry2, _rs=rs, _rd=rd, _off=off):
                lanes = g * LANES + lax.iota(jnp.int32, LANES)

                def word_body(kb, carry3):
                    kv, f0, f1, f2, f3 = carry3
                    pz = jnp.zeros((2 * LANES,), jnp.bfloat16)
                    pacc = [pz, pz]
                    for j in range(16):
                        ws = plsc.load_gather(_rs, [lanes, kv])
                        wd = plsc.load_gather(_rd, [lanes, kv])
                        p = (plsc.bitcast(ws, jnp.bfloat16) *
                             plsc.bitcast(wd, jnp.bfloat16))
                        pacc[j % 2] = pacc[j % 2] + p
                        kv = (kv + 1) & (DW - 1)
                    a0, b0 = plsc.unpack(pacc[0],
                                         format=plsc.PackFormat.INTERLEAVED)
                    a1, b1 = plsc.unpack(pacc[1],
                                         format=plsc.PackFormat.INTERLEAVED)
                    return (kv, f0 + a0, f1 + b0, f2 + a1, f3 + b1)

                zero = jnp.zeros((LANES,), jnp.float32)
                kv0 = lax.iota(jnp.int32, LANES) & (DW - 1)
                carry3 = lax.fori_loop(
                    0, DW // 16, word_body, (kv0, zero, zero, zero, zero))
                _, f0, f1, f2, f3 = carry3
                out_full[pl.ds(_off + g * LANES, LANES)] = (f0 + f1) + (f2 + f3)
                return carry2

            lax.fori_loop(0, C // LANES, group_body, 0)

            @pl.when(c + 2 < NCH)
            def _():
                issue(c + 2, b)

        return carry

    lax.fori_loop(0, (NCH + 1) // 2, pair_body, 0)
    pltpu.sync_copy(out_full, out_hbm.at[pl.ds(wbase, PER_W)])


@jax.jit
def _link_predict(zw, src, dst):
    mesh = plsc.VectorSubcoreMesh(core_axis_name="c", subcore_axis_name="s")
    run = pl.kernel(
        _sc_body,
        out_type=jax.ShapeDtypeStruct((N_EDGE,), jnp.float32),
        mesh=mesh,
        scratch_types=[
            pltpu.VMEM((PER_W,), jnp.int32),
            pltpu.VMEM((PER_W,), jnp.int32),
            pltpu.VMEM((PER_W,), jnp.float32),
            pltpu.VMEM_SHARED((N_NODE, DW), jnp.int32),
            pltpu.VMEM((C, DW), jnp.int32),
            pltpu.VMEM((C, DW), jnp.int32),
            pltpu.VMEM((C, DW), jnp.int32),
            pltpu.VMEM((C, DW), jnp.int32),
            pltpu.SemaphoreType.DMA,
            pltpu.SemaphoreType.DMA,
            pltpu.SemaphoreType.DMA,
            pltpu.SemaphoreType.DMA,
        ],
        compiler_params=pltpu.CompilerParams(
            use_tc_tiling_on_sc=False, needs_layout_passes=False),
    )
    return run(zw, src, dst)


def kernel(z, edge):
    src = edge[:, 0].astype(jnp.int32)
    dst = edge[:, 1].astype(jnp.int32)
    zw = jax.lax.bitcast_convert_type(
        z.astype(jnp.bfloat16).reshape(N_NODE, DW, 2), jnp.int32)
    return _link_predict(zw, src, dst)


# speedup vs baseline: 1.2044x; 1.1708x over previous
import jax
import jax.numpy as jnp
from jax import lax
from jax.experimental import pallas as pl
from jax.experimental.pallas import tpu as pltpu
from jax.experimental.pallas import tpu_sc as plsc

N_EDGE = 160000
D = 256
NC = 2
NS = 16
NW = NC * NS
PER_W = N_EDGE // NW
C = 96
N_FULL = PER_W // C
TAIL_OFF = PER_W - C
NCH = N_FULL + 2
LANES = 16


def _sc_body(z_hbm, src_hbm, dst_hbm, out_hbm,
             srcv, dstv, out_full,
             rows_s0, rows_d0, rows_s1, rows_d1,
             sem_s0, sem_d0, sem_s1, sem_d1):
    wid = lax.axis_index("s") * NC + lax.axis_index("c")
    wbase = wid * PER_W
    rows_s = (rows_s0, rows_s1)
    rows_d = (rows_d0, rows_d1)
    sem_s = (sem_s0, sem_s1)
    sem_d = (sem_d0, sem_d1)

    pltpu.sync_copy(src_hbm.at[pl.ds(wbase, PER_W)], srcv)
    pltpu.sync_copy(dst_hbm.at[pl.ds(wbase, PER_W)], dstv)

    def chunk_off(c):
        return jnp.where(c < N_FULL, c * C, TAIL_OFF)

    def issue(c, b):
        off = chunk_off(c)
        pltpu.async_copy(z_hbm.at[srcv.at[pl.ds(off, C)]], rows_s[b],
                         sem_s[b])
        pltpu.async_copy(z_hbm.at[dstv.at[pl.ds(off, C)]], rows_d[b],
                         sem_d[b])

    def drain(c, b):
        off = chunk_off(c)
        pltpu.make_async_copy(z_hbm.at[srcv.at[pl.ds(off, C)]], rows_s[b],
                              sem_s[b]).wait()
        pltpu.make_async_copy(z_hbm.at[dstv.at[pl.ds(off, C)]], rows_d[b],
                              sem_d[b]).wait()

    for b in range(2):
        issue(b, b)

    def pair_body(i, carry):
        for b in range(2):
            c = i * 2 + b
            drain(c, b)
            rs, rd = rows_s[b], rows_d[b]
            off = chunk_off(c)

            def group_body(g, carry2, _rs=rs, _rd=rd, _off=off):
                lanes = g * LANES + lax.iota(jnp.int32, LANES)

                def feat_body(kb, carry3):
                    kv, a0, a1, a2, a3 = carry3
                    accs = [a0, a1, a2, a3]
                    for j in range(16):
                        vs = plsc.load_gather(_rs, [lanes, kv])
                        vd = plsc.load_gather(_rd, [lanes, kv])
                        accs[j % 4] = accs[j % 4] + vs * vd
                        kv = (kv + 1) & (D - 1)
                    return (kv, *accs)

                zero = jnp.zeros((LANES,), jnp.float32)
                kv0 = lax.iota(jnp.int32, LANES)
                _, a0, a1, a2, a3 = lax.fori_loop(
                    0, D // 16, feat_body, (kv0, zero, zero, zero, zero))
                out_full[pl.ds(_off + g * LANES, LANES)] = (a0 + a1) + (a2 + a3)
                return carry2

            lax.fori_loop(0, C // LANES, group_body, 0)

            @pl.when(c + 2 < NCH)
            def _():
                issue(c + 2, b)

        return carry

    lax.fori_loop(0, NCH // 2, pair_body, 0)
    pltpu.sync_copy(out_full, out_hbm.at[pl.ds(wbase, PER_W)])


@jax.jit
def _link_predict(z, src, dst):
    mesh = plsc.VectorSubcoreMesh(core_axis_name="c", subcore_axis_name="s")
    run = pl.kernel(
        _sc_body,
        out_type=jax.ShapeDtypeStruct((N_EDGE,), jnp.float32),
        mesh=mesh,
        scratch_types=[
            pltpu.VMEM((PER_W,), jnp.int32),
            pltpu.VMEM((PER_W,), jnp.int32),
            pltpu.VMEM((PER_W,), jnp.float32),
            pltpu.VMEM((C, D), jnp.float32),
            pltpu.VMEM((C, D), jnp.float32),
            pltpu.VMEM((C, D), jnp.float32),
            pltpu.VMEM((C, D), jnp.float32),
            pltpu.SemaphoreType.DMA,
            pltpu.SemaphoreType.DMA,
            pltpu.SemaphoreType.DMA,
            pltpu.SemaphoreType.DMA,
        ],
        compiler_params=pltpu.CompilerParams(
            use_tc_tiling_on_sc=False, needs_layout_passes=False),
    )
    return run(z, src, dst)


def kernel(z, edge):
    src = edge[:, 0].astype(jnp.int32)
    dst = edge[:, 1].astype(jnp.int32)
    return _link_predict(z, src, dst)


# speedup vs baseline: 1.2100x; 1.0047x over previous
import jax
import jax.numpy as jnp
from jax import lax
from jax.experimental import pallas as pl
from jax.experimental.pallas import tpu as pltpu
from jax.experimental.pallas import tpu_sc as plsc

N_EDGE = 160000
D = 256
NC = 2
NS = 16
NW = NC * NS
PER_W = N_EDGE // NW
C = 112
N_FULL = PER_W // C
TAIL_OFF = PER_W - C
NCH = N_FULL + 2
LANES = 16


def _sc_body(z_hbm, src_hbm, dst_hbm, out_hbm,
             srcv, dstv, out_full,
             rows_s0, rows_d0, rows_s1, rows_d1,
             sem_s0, sem_d0, sem_s1, sem_d1):
    wid = lax.axis_index("s") * NC + lax.axis_index("c")
    wbase = wid * PER_W
    rows_s = (rows_s0, rows_s1)
    rows_d = (rows_d0, rows_d1)
    sem_s = (sem_s0, sem_s1)
    sem_d = (sem_d0, sem_d1)

    pltpu.sync_copy(src_hbm.at[pl.ds(wbase, PER_W)], srcv)
    pltpu.sync_copy(dst_hbm.at[pl.ds(wbase, PER_W)], dstv)

    def chunk_off(c):
        return jnp.where(c < N_FULL, c * C, TAIL_OFF)

    def issue(c, b):
        off = chunk_off(c)
        pltpu.async_copy(z_hbm.at[srcv.at[pl.ds(off, C)]], rows_s[b],
                         sem_s[b])
        pltpu.async_copy(z_hbm.at[dstv.at[pl.ds(off, C)]], rows_d[b],
                         sem_d[b])

    def drain(c, b):
        off = chunk_off(c)
        pltpu.make_async_copy(z_hbm.at[srcv.at[pl.ds(off, C)]], rows_s[b],
                              sem_s[b]).wait()
        pltpu.make_async_copy(z_hbm.at[dstv.at[pl.ds(off, C)]], rows_d[b],
                              sem_d[b]).wait()

    for b in range(2):
        issue(b, b)

    def pair_body(i, carry):
        for b in range(2):
            c = i * 2 + b
            drain(c, b)
            rs, rd = rows_s[b], rows_d[b]
            off = chunk_off(c)

            def group_body(g, carry2, _rs=rs, _rd=rd, _off=off):
                lanes = g * LANES + lax.iota(jnp.int32, LANES)

                def feat_body(kb, carry3):
                    kv, a0, a1, a2, a3 = carry3
                    accs = [a0, a1, a2, a3]
                    for j in range(16):
                        vs = plsc.load_gather(_rs, [lanes, kv])
                        vd = plsc.load_gather(_rd, [lanes, kv])
                        accs[j % 4] = accs[j % 4] + vs * vd
                        kv = (kv + 1) & (D - 1)
                    return (kv, *accs)

                zero = jnp.zeros((LANES,), jnp.float32)
                kv0 = lax.iota(jnp.int32, LANES)
                _, a0, a1, a2, a3 = lax.fori_loop(
                    0, D // 16, feat_body, (kv0, zero, zero, zero, zero))
                out_full[pl.ds(_off + g * LANES, LANES)] = (a0 + a1) + (a2 + a3)
                return carry2

            lax.fori_loop(0, C // LANES, group_body, 0)

            @pl.when(c + 2 < NCH)
            def _():
                issue(c + 2, b)

        return carry

    lax.fori_loop(0, NCH // 2, pair_body, 0)
    pltpu.sync_copy(out_full, out_hbm.at[pl.ds(wbase, PER_W)])


@jax.jit
def _link_predict(z, src, dst):
    mesh = plsc.VectorSubcoreMesh(core_axis_name="c", subcore_axis_name="s")
    run = pl.kernel(
        _sc_body,
        out_type=jax.ShapeDtypeStruct((N_EDGE,), jnp.float32),
        mesh=mesh,
        scratch_types=[
            pltpu.VMEM((PER_W,), jnp.int32),
            pltpu.VMEM((PER_W,), jnp.int32),
            pltpu.VMEM((PER_W,), jnp.float32),
            pltpu.VMEM((C, D), jnp.float32),
            pltpu.VMEM((C, D), jnp.float32),
            pltpu.VMEM((C, D), jnp.float32),
            pltpu.VMEM((C, D), jnp.float32),
            pltpu.SemaphoreType.DMA,
            pltpu.SemaphoreType.DMA,
            pltpu.SemaphoreType.DMA,
            pltpu.SemaphoreType.DMA,
        ],
        compiler_params=pltpu.CompilerParams(
            use_tc_tiling_on_sc=False, needs_layout_passes=False),
    )
    return run(z, src, dst)


def kernel(z, edge):
    src = edge[:, 0].astype(jnp.int32)
    dst = edge[:, 1].astype(jnp.int32)
    return _link_predict(z, src, dst)
